# Initial kernel scaffold; baseline (speedup 1.0000x reference)
import jax
import jax.numpy as jnp
from jax.experimental import pallas as pl


def kernel(x_nchw, conv1_w, conv1_b, conv2_w, conv2_b, conv3_w, conv3_b, conv4_w, conv4_b, fc1_w, fc1_b, fc2_w, fc2_b, fc3_w, fc3_b):
    raise NotImplementedError("write your pallas kernel here")


# speedup vs baseline: 10.2078x; 10.2078x over previous
import jax
import jax.numpy as jnp
from jax.experimental import pallas as pl
from jax.experimental.pallas import tpu as pltpu

KH = 5
H_IN, W_IN = 45, 12
H_PAD = 48
C = 64
LANES = 4 * C
NCH1, NCH2, NCH3, NCH4 = 11, 10, 9, 8
BN = 128
FEAT_PAD = W_IN * NCH4 * LANES
VMEM_LIMIT = 48 * 1024 * 1024

_BF = jnp.bfloat16
_F32 = jnp.float32


def _conv1_weight(w1):
    w8 = jnp.reshape(
        jnp.stack([jnp.pad(w1, ((d, 3 - d), (0, 0))) for d in range(4)], axis=1),
        (8, LANES))
    return jnp.concatenate(
        [jnp.pad(w8, ((4 * j, 40 - 4 * j), (0, 0))) for j in range(NCH1)], axis=1)


def _tap_mats(w):
    w5 = w.reshape(KH, C, C)
    t = jnp.stack([jnp.pad(w5, ((d, 3 - d), (0, 0), (0, 0))) for d in range(4)],
                  axis=2).reshape(2 * LANES, LANES)
    return t[:LANES], t[LANES:]


def _fused_kernel(x_ref, w1_ref, b1_ref, w2a_ref, w2b_ref, b2_ref,
                  w3a_ref, w3b_ref, b3_ref, w4a_ref, w4b_ref, b4_ref,
                  f1w_ref, f1b_ref, f2w_ref, f2b_ref, f3w_ref, f3b_ref,
                  o_ref):
    bn = x_ref.shape[1]
    s = W_IN * bn
    xb = x_ref[...].reshape(s, H_PAD).astype(_BF)

    b1 = b1_ref[...]
    acts = []
    for j in range(NCH1):
        z = jnp.dot(xb, w1_ref[:, j * LANES:(j + 1) * LANES],
                    preferred_element_type=_F32)
        acts.append(jnp.maximum(z + b1, 0.0).astype(_BF))

    for wa_ref, wb_ref, b_ref, n_out in ((w2a_ref, w2b_ref, b2_ref, NCH2),
                                         (w3a_ref, w3b_ref, b3_ref, NCH3),
                                         (w4a_ref, w4b_ref, b4_ref, NCH4)):
        wa, wb, b = wa_ref[...], wb_ref[...], b_ref[...]
        nxt = []
        for j in range(n_out):
            z = (jnp.dot(acts[j], wa, preferred_element_type=_F32)
                 + jnp.dot(acts[j + 1], wb, preferred_element_type=_F32))
            nxt.append(jnp.maximum(z + b, 0.0).astype(_BF))
        acts = nxt

    acc = None
    for w in range(W_IN):
        for j in range(NCH4):
            lhs = acts[j][w * bn:(w + 1) * bn, :]
            rhs = f1w_ref[(w * NCH4 + j) * LANES:(w * NCH4 + j + 1) * LANES, :]
            d = jnp.dot(lhs, rhs, preferred_element_type=_F32)
            acc = d if acc is None else acc + d
    h = jnp.maximum(acc + f1b_ref[...], 0.0).astype(_BF)
    h = jnp.maximum(jnp.dot(h, f2w_ref[...], preferred_element_type=_F32)
                    + f2b_ref[...], 0.0).astype(_BF)
    o_ref[...] = (jnp.dot(h, f3w_ref[...], preferred_element_type=_F32)
                  + f3b_ref[...])


def kernel(x_nchw, conv1_w, conv1_b, conv2_w, conv2_b, conv3_w, conv3_b,
           conv4_w, conv4_b, fc1_w, fc1_b, fc2_w, fc2_b, fc3_w, fc3_b):
    n = x_nchw.shape[0]

    xt = jnp.pad(jnp.transpose(x_nchw[:, 0], (2, 0, 1)),
                 ((0, 0), (0, 0), (0, H_PAD - H_IN)))

    w1 = _conv1_weight(conv1_w).astype(_BF)
    w2a, w2b = _tap_mats(conv2_w)
    w3a, w3b = _tap_mats(conv3_w)
    w4a, w4b = _tap_mats(conv4_w)

    f1 = fc1_w.reshape(29, W_IN, C, 128).transpose(1, 0, 2, 3)
    f1 = jnp.pad(f1, ((0, 0), (0, 3), (0, 0), (0, 0))).reshape(FEAT_PAD, 128)

    const = lambda i: (0, 0)
    grid = (n // BN,)
    out = pl.pallas_call(
        _fused_kernel,
        out_shape=jax.ShapeDtypeStruct((n, 22), _F32),
        grid=grid,
        in_specs=[
            pl.BlockSpec((W_IN, BN, H_PAD), lambda i: (0, i, 0)),
            pl.BlockSpec((H_PAD, NCH1 * LANES), const),
            pl.BlockSpec((1, LANES), const),
            pl.BlockSpec((LANES, LANES), const),
            pl.BlockSpec((LANES, LANES), const),
            pl.BlockSpec((1, LANES), const),
            pl.BlockSpec((LANES, LANES), const),
            pl.BlockSpec((LANES, LANES), const),
            pl.BlockSpec((1, LANES), const),
            pl.BlockSpec((LANES, LANES), const),
            pl.BlockSpec((LANES, LANES), const),
            pl.BlockSpec((1, LANES), const),
            pl.BlockSpec((FEAT_PAD, 128), const),
            pl.BlockSpec((1, 128), const),
            pl.BlockSpec((128, 128), const),
            pl.BlockSpec((1, 128), const),
            pl.BlockSpec((128, 22), const),
            pl.BlockSpec((1, 22), const),
        ],
        out_specs=pl.BlockSpec((BN, 22), lambda i: (i, 0)),
        compiler_params=pltpu.CompilerParams(
            dimension_semantics=("parallel",),
            vmem_limit_bytes=VMEM_LIMIT),
    )(xt,
      w1, jnp.tile(conv1_b, (1, 4)),
      w2a.astype(_BF), w2b.astype(_BF), jnp.tile(conv2_b, (1, 4)),
      w3a.astype(_BF), w3b.astype(_BF), jnp.tile(conv3_b, (1, 4)),
      w4a.astype(_BF), w4b.astype(_BF), jnp.tile(conv4_b, (1, 4)),
      f1.astype(_BF), fc1_b,
      fc2_w.astype(_BF), fc2_b,
      fc3_w.astype(_BF), fc3_b)
    return out
